# Initial kernel scaffold; baseline (speedup 1.0000x reference)
import jax
import jax.numpy as jnp
from jax.experimental import pallas as pl


def kernel(img, coords, Wg, bg, W0, b0, W1, b1, W2, b2, W3, b3, W4, b4, top_k):
    raise NotImplementedError("write your pallas kernel here")


# speedup vs baseline: 1.1628x; 1.1628x over previous
import functools

import jax
import jax.numpy as jnp
from jax import lax
from jax.experimental import pallas as pl
from jax.experimental.pallas import tpu as pltpu

_NUM_EXPS = [8, 16, 64, 256, 1024]
_KS = [4, 4, 32, 32, 256]
_OFFS = [0, 8, 24, 88, 344, 1368]
_HID = 256
_OUT = 3
_NB = 2
_NC = 256
_GIN = 3072
_GOUT = 1368
_GATE_TILE = 128
_GATE_STEPS = 11


def _topk_mask_gates(g, k):
    ne = g.shape[1]
    bits = lax.bitcast_convert_type(jnp.abs(g), jnp.int32)

    def bs_body(_, carry):
        lo, hi = carry
        mid = lo + (hi - lo + 1) // 2
        cnt = jnp.sum((bits >= mid).astype(jnp.int32), axis=1, keepdims=True)
        ok = cnt >= k
        return jnp.where(ok, mid, lo), jnp.where(ok, hi, mid - 1)

    lo0 = jnp.zeros((_NB, 1), jnp.int32)
    hi0 = jnp.full((_NB, 1), 0x7F800000, jnp.int32)
    t, _ = lax.fori_loop(0, 32, bs_body, (lo0, hi0))

    gt = bits > t
    eq = bits == t
    c1 = jnp.sum(gt.astype(jnp.int32), axis=1, keepdims=True)
    row = lax.broadcasted_iota(jnp.int32, (ne, ne), 0)
    col = lax.broadcasted_iota(jnp.int32, (ne, ne), 1)
    lt = (row <= col).astype(jnp.float32)
    rank = jnp.dot(eq.astype(jnp.float32), lt,
                   preferred_element_type=jnp.float32)
    mask = gt | (eq & (rank <= (k - c1).astype(jnp.float32)))
    return jnp.where(mask, g, 0.0)


def _gate_body(topk_ref, x_ref, wg_ref, bg_ref, out_ref, raw_ref):
    j = pl.program_id(0)
    raw_ref[:, pl.ds(j * _GATE_TILE, _GATE_TILE)] = jnp.dot(
        x_ref[...], wg_ref[...], preferred_element_type=jnp.float32)

    @pl.when(j == _GATE_STEPS - 1)
    def _epilogue():
        use_topk = topk_ref[0, 0] != 0
        for li in range(5):
            off, ne, k = _OFFS[li], _NUM_EXPS[li], _KS[li]
            g = raw_ref[:, off:off + ne] + bg_ref[:, off:off + ne]
            z = _topk_mask_gates(g, k)
            nrm = jnp.sqrt(jnp.sum(g * g, axis=1, keepdims=True))
            gn = g / jnp.maximum(nrm, 1e-12)
            out_ref[:, off:off + ne] = jnp.where(use_topk, z, gn)


def _compute_gates(x_img, Wg, bg, topk_s):
    return pl.pallas_call(
        _gate_body,
        grid=(_GATE_STEPS,),
        in_specs=[
            pl.BlockSpec(memory_space=pltpu.SMEM),
            pl.BlockSpec((_NB, _GIN), lambda j: (0, 0)),
            pl.BlockSpec((_GIN, _GATE_TILE), lambda j: (0, j)),
            pl.BlockSpec((1, _GOUT), lambda j: (0, 0)),
        ],
        out_specs=pl.BlockSpec((_NB, _GOUT), lambda j: (0, 0)),
        out_shape=jax.ShapeDtypeStruct((_NB, _GOUT), jnp.float32),
        scratch_shapes=[pltpu.VMEM((_NB, _GATE_TILE * _GATE_STEPS),
                                   jnp.float32)],
    )(topk_s, x_img, Wg, bg)


def _gate_col(g_row, tile, ne):
    jj = lax.broadcasted_iota(jnp.int32, (tile, ne), 0)
    ee = lax.broadcasted_iota(jnp.int32, (tile, ne), 1)
    sel = (jj % ne) == ee
    return jnp.sum(jnp.where(sel, jnp.broadcast_to(g_row, (tile, ne)), 0.0),
                   axis=1, keepdims=True)


def _combine_matmul(yb, g_row, ne, h):
    tile = h * ne
    gcol = _gate_col(g_row, tile, ne)
    jj = lax.broadcasted_iota(jnp.int32, (tile, h), 0)
    dd = lax.broadcasted_iota(jnp.int32, (tile, h), 1)
    gmat = jnp.where(jj // ne == dd, jnp.broadcast_to(gcol, (tile, h)), 0.0)
    return jnp.dot(yb, gmat, preferred_element_type=jnp.float32)


def _l0_body(coords_ref, w_ref, b_ref, g_ref, out_ref):
    y = jnp.sin(30.0 * (jnp.dot(coords_ref[...], w_ref[...],
                                preferred_element_type=jnp.float32)
                        + b_ref[...]))
    for b in range(_NB):
        out_ref[pl.ds(b * _NC, _NC), :] = _combine_matmul(
            y, g_ref[b:b + 1, :], _NUM_EXPS[0], _HID)


def _layer0(coords, W0, b0, g0):
    return pl.pallas_call(
        _l0_body,
        out_shape=jax.ShapeDtypeStruct((_NB * _NC, _HID), jnp.float32),
    )(coords, W0, b0, g0)


def _mid_body(x_ref, w_ref, b_ref, g_ref, out_ref, *, ne, tile, do_sin):
    h = tile // ne
    u = jnp.dot(x_ref[...], w_ref[...],
                preferred_element_type=jnp.float32) + b_ref[...]
    y = jnp.sin(30.0 * u) if do_sin else u
    for b in range(_NB):
        out_ref[0, pl.ds(b * _NC, _NC), :] = _combine_matmul(
            y[b * _NC:(b + 1) * _NC, :], g_ref[b:b + 1, :], ne, h)


def _mid_layer(x, W, b, g, *, ne, d, tile, do_sin):
    ncols = W.shape[1]
    steps = ncols // tile
    h = tile // ne
    body = functools.partial(_mid_body, ne=ne, tile=tile, do_sin=do_sin)
    out3 = pl.pallas_call(
        body,
        grid=(steps,),
        in_specs=[
            pl.BlockSpec((_NB * _NC, _HID), lambda j: (0, 0)),
            pl.BlockSpec((_HID, tile), lambda j: (0, j)),
            pl.BlockSpec((1, tile), lambda j: (0, j)),
            pl.BlockSpec((_NB, ne), lambda j: (0, 0)),
        ],
        out_specs=pl.BlockSpec((1, _NB * _NC, h), lambda j: (j, 0, 0)),
        out_shape=jax.ShapeDtypeStruct((steps, _NB * _NC, h), jnp.float32),
    )(x, W, b, g)
    return out3.transpose(1, 0, 2).reshape(_NB * _NC, d)


def _l4_body(x_ref, w_ref, b_ref, g_ref, out_ref):
    ne = _NUM_EXPS[4]
    u = jnp.dot(x_ref[...], w_ref[...],
                preferred_element_type=jnp.float32) + b_ref[...]
    for b in range(_NB):
        out_ref[b, :, :] = _combine_matmul(
            u[b * _NC:(b + 1) * _NC, :], g_ref[b:b + 1, :], ne, _OUT)


def _layer4(x, W4, b4, g4):
    return pl.pallas_call(
        _l4_body,
        out_shape=jax.ShapeDtypeStruct((_NB, _NC, _OUT), jnp.float32),
    )(x, W4, b4, g4)


def kernel(img, coords, Wg, bg, W0, b0, W1, b1, W2, b2, W3, b3, W4, b4,
           top_k):
    x_img = img.reshape(_NB, _GIN)
    topk_s = jnp.asarray(top_k, jnp.int32).reshape(1, 1)
    gates = _compute_gates(x_img, Wg, bg.reshape(1, _GOUT), topk_s)

    g0 = gates[:, _OFFS[0]:_OFFS[1]]
    g1 = gates[:, _OFFS[1]:_OFFS[2]]
    g2 = gates[:, _OFFS[2]:_OFFS[3]]
    g3 = gates[:, _OFFS[3]:_OFFS[4]]
    g4 = gates[:, _OFFS[4]:_OFFS[5]]

    x = _layer0(coords, W0, b0.reshape(1, -1), g0)
    x = _mid_layer(x, W1, b1.reshape(1, -1), g1,
                   ne=16, d=_HID, tile=2048, do_sin=True)
    x = _mid_layer(x, W2, b2.reshape(1, -1), g2,
                   ne=64, d=_HID, tile=2048, do_sin=True)
    x = _mid_layer(x, W3, b3.reshape(1, -1), g3,
                   ne=256, d=_HID, tile=2048, do_sin=True)
    return _layer4(x, W4, b4.reshape(1, -1), g4)
